# baseline (speedup 1.0000x reference)
import jax
import jax.numpy as jnp
from jax import lax
from jax.experimental import pallas as pl
from jax.experimental.pallas import tpu as pltpu
from jax.experimental.pallas import tpu_sc as plsc

BATCH = 1024
MAXLEN = 200
EMBED = 64

NUM_CORES = 2
NUM_SUBCORES = 16
NUM_WORKERS = NUM_CORES * NUM_SUBCORES
SEQS_PER_WORKER = BATCH // NUM_WORKERS
LANES = 16
NBUF = 4
LAG = NBUF - 1
SPLIT = 72
RTILES = MAXLEN // 8


def _gather_kernel(x2_hbm, tok_hbm, out_hbm, xslab, idx_v,
                   gbufs, gsems, osems):
    wid = lax.axis_index("s") * NUM_CORES + lax.axis_index("c")
    base = wid * SEQS_PER_WORKER

    btile = wid // 4
    lo = 32 * (wid % 4)
    for r_t in range(RTILES):
        pltpu.make_async_copy(
            x2_hbm.at[pl.ds(r_t * 64 + btile * 8, 8), pl.ds(lo, 32)],
            xslab.at[pl.ds(r_t * 8, 8)], gsems[0]
        ).start()
    for r_t in range(RTILES):
        pltpu.make_async_copy(
            x2_hbm.at[pl.ds(r_t * 64 + btile * 8, 8), pl.ds(lo, 32)],
            xslab.at[pl.ds(r_t * 8, 8)], gsems[0]
        ).wait()

    @pl.loop(0, SEQS_PER_WORKER)
    def _(s):
        col = jnp.full((LANES,), s, jnp.int32)
        for c in range(13):
            rows = c * LANES + lax.iota(jnp.int32, LANES)
            idx_v[s, pl.ds(c * LANES, LANES)] = plsc.load_gather(
                xslab, [rows, col])

    def start_gather(s, b):
        pltpu.make_async_copy(
            tok_hbm.at[idx_v.at[s, pl.ds(0, SPLIT)]],
            gbufs[b].at[pl.ds(0, SPLIT)], gsems[b]
        ).start()
        pltpu.make_async_copy(
            tok_hbm.at[idx_v.at[s, pl.ds(SPLIT, MAXLEN - SPLIT)]],
            gbufs[b].at[pl.ds(SPLIT, MAXLEN - SPLIT)], gsems[b]
        ).start()

    def wait_gather(s, b):
        pltpu.make_async_copy(
            tok_hbm.at[idx_v.at[s, pl.ds(0, SPLIT)]],
            gbufs[b].at[pl.ds(0, SPLIT)], gsems[b]
        ).wait()
        pltpu.make_async_copy(
            tok_hbm.at[idx_v.at[s, pl.ds(SPLIT, MAXLEN - SPLIT)]],
            gbufs[b].at[pl.ds(SPLIT, MAXLEN - SPLIT)], gsems[b]
        ).wait()

    def start_write(s, b):
        pltpu.make_async_copy(gbufs[b], out_hbm.at[base + s], osems[b]).start()

    def wait_write(s, b):
        pltpu.make_async_copy(gbufs[b], out_hbm.at[base + s], osems[b]).wait()

    for b in range(NBUF):
        start_gather(b, b)
    wait_gather(0, 0)
    start_write(0, 0)

    @pl.loop(1, SEQS_PER_WORKER // NBUF)
    def _(g):
        for b in range(NBUF):
            k = g * NBUF + b
            wait_write(k - NBUF, b)
            start_gather(k, b)
            kb = k - LAG
            bb = (b + 1) % NBUF
            wait_gather(kb, bb)
            start_write(kb, bb)

    for kb in range(SEQS_PER_WORKER - LAG, SEQS_PER_WORKER):
        bb = kb % NBUF
        wait_gather(kb, bb)
        start_write(kb, bb)
    for kb in range(SEQS_PER_WORKER - NBUF, SEQS_PER_WORKER):
        wait_write(kb, kb % NBUF)


def _sc_gather(x2, token_table):
    mesh = plsc.VectorSubcoreMesh(core_axis_name="c", subcore_axis_name="s")
    gbuf = lambda: pltpu.VMEM((MAXLEN, EMBED), jnp.float32)

    def body(x_hbm, tok_hbm, out_hbm, xslab, idx_v, g0, g1, g2, g3,
             gs0, gs1, gs2, gs3, os0, os1, os2, os3):
        _gather_kernel(x_hbm, tok_hbm, out_hbm, xslab, idx_v,
                       (g0, g1, g2, g3), (gs0, gs1, gs2, gs3),
                       (os0, os1, os2, os3))

    k = pl.kernel(
        body,
        out_type=jax.ShapeDtypeStruct((BATCH, MAXLEN, EMBED), jnp.float32),
        mesh=mesh,
        scratch_types=[
            pltpu.VMEM((208, SEQS_PER_WORKER), jnp.int32),
            pltpu.VMEM((SEQS_PER_WORKER, 208), jnp.int32),
            gbuf(), gbuf(), gbuf(), gbuf(),
            pltpu.SemaphoreType.DMA, pltpu.SemaphoreType.DMA,
            pltpu.SemaphoreType.DMA, pltpu.SemaphoreType.DMA,
            pltpu.SemaphoreType.DMA, pltpu.SemaphoreType.DMA,
            pltpu.SemaphoreType.DMA, pltpu.SemaphoreType.DMA,
        ],
        compiler_params=pltpu.CompilerParams(use_tc_tiling_on_sc=False,
                                             needs_layout_passes=False),
    )
    return k(x2, token_table)


B_BLK = 128


def _xpose_body(tok_ref, post_ref, out_ref):
    for r in range(MAXLEN):
        posvec = post_ref[:, r]
        out_ref[pl.ds(r * EMBED, EMBED), :] = (
            tok_ref[:, r, :].T + posvec[:, None])


def _tc_xpose_add(tok, post):
    return pl.pallas_call(
        _xpose_body,
        out_shape=jax.ShapeDtypeStruct((MAXLEN * EMBED, BATCH), jnp.float32),
        grid=(BATCH // B_BLK,),
        in_specs=[
            pl.BlockSpec((B_BLK, MAXLEN, EMBED), lambda i: (i, 0, 0)),
            pl.BlockSpec((EMBED, MAXLEN), lambda i: (0, 0)),
        ],
        out_specs=pl.BlockSpec((MAXLEN * EMBED, B_BLK), lambda i: (0, i)),
    )(tok, post)


@jax.jit
def kernel(x, token_table, pos_table):
    x2 = (x.astype(jnp.int32).T.reshape(RTILES, 8, 8, 128)
          .transpose(0, 2, 1, 3).reshape(8 * MAXLEN, 128))
    tok = _sc_gather(x2, token_table)
    out_t = _tc_xpose_add(tok, pos_table.T)
    return jnp.transpose(out_t.reshape(MAXLEN, EMBED, BATCH), (2, 0, 1))


# speedup vs baseline: 1.0320x; 1.0320x over previous
import jax
import jax.numpy as jnp
from jax import lax
from jax.experimental import pallas as pl
from jax.experimental.pallas import tpu as pltpu
from jax.experimental.pallas import tpu_sc as plsc

BATCH = 1024
MAXLEN = 200
EMBED = 64
HALF = 100
H_PAD = 104

NUM_CORES = 2
NUM_SUBCORES = 16
NUM_WORKERS = NUM_CORES * NUM_SUBCORES
SEQS_PER_WORKER = BATCH // NUM_WORKERS
LANES = 16
NBUF = 2
NROUNDS = SEQS_PER_WORKER // NBUF


def _embed_kernel(x_hbm, tok_hbm, out_hbm, idx_v,
                  gbufs, wbufs, gsems, osems):
    wid = lax.axis_index("s") * NUM_CORES + lax.axis_index("c")
    base = wid * SEQS_PER_WORKER

    pltpu.sync_copy(x_hbm.at[pl.ds(base, SEQS_PER_WORKER)], idx_v)

    def start_gather(s, b):
        pltpu.make_async_copy(
            tok_hbm.at[idx_v.at[s, 0]], gbufs[b].at[pl.ds(0, HALF)], gsems[b]
        ).start()
        pltpu.make_async_copy(
            tok_hbm.at[idx_v.at[s, 1]], gbufs[b].at[pl.ds(HALF, HALF)], gsems[b]
        ).start()

    def wait_gather(s, b):
        pltpu.make_async_copy(
            tok_hbm.at[idx_v.at[s, 0]], gbufs[b].at[pl.ds(0, HALF)], gsems[b]
        ).wait()
        pltpu.make_async_copy(
            tok_hbm.at[idx_v.at[s, 1]], gbufs[b].at[pl.ds(HALF, HALF)], gsems[b]
        ).wait()

    def add_pos(b):
        @pl.loop(0, HALF)
        def _(h):
            r = 2 * h
            for j in range(EMBED // LANES):
                c = pl.ds(j * LANES, LANES)
                cl = pl.ds(j * LANES + EMBED, LANES)
                wbufs[b][h, c] = gbufs[b][r, c]
                wbufs[b][h, cl] = gbufs[b][r + 1, c]

    def start_write(s, b):
        pltpu.make_async_copy(
            wbufs[b], out_hbm.at[base + s, pl.ds(0, HALF)], osems[b]
        ).start()

    def wait_write(s, b):
        pltpu.make_async_copy(
            wbufs[b], out_hbm.at[base + s, pl.ds(0, HALF)], osems[b]
        ).wait()

    for b in range(NBUF):
        start_gather(b, b)

    for b in range(NBUF):
        wait_gather(b, b)
        add_pos(b)
        start_gather(NBUF + b, b)
        start_write(b, b)

    @pl.loop(1, NROUNDS - 1)
    def _(g):
        for b in range(NBUF):
            s = g * NBUF + b
            wait_gather(s, b)
            wait_write(s - NBUF, b)
            add_pos(b)
            start_gather(s + NBUF, b)
            start_write(s, b)

    for b in range(NBUF):
        s = (NROUNDS - 1) * NBUF + b
        wait_gather(s, b)
        wait_write(s - NBUF, b)
        add_pos(b)
        start_write(s, b)
    for b in range(NBUF):
        s = (NROUNDS - 1) * NBUF + b
        wait_write(s, b)


def _wrapped(x3, token_table):
    mesh = plsc.VectorSubcoreMesh(core_axis_name="c", subcore_axis_name="s")
    vmem_rows = lambda: pltpu.VMEM((MAXLEN, EMBED), jnp.float32)

    def body(x_hbm, tok_hbm, out_hbm, idx_v,
             g0, g1, w0, w1, gs0, gs1, os0, os1):
        _embed_kernel(x_hbm, tok_hbm, out_hbm, idx_v,
                      (g0, g1), (w0, w1), (gs0, gs1), (os0, os1))

    k = pl.kernel(
        body,
        out_type=jax.ShapeDtypeStruct((BATCH, H_PAD, 2 * EMBED), jnp.float32),
        mesh=mesh,
        scratch_types=[
            pltpu.VMEM((SEQS_PER_WORKER, 2, HALF), jnp.int32),
            vmem_rows(), vmem_rows(),
            pltpu.VMEM((HALF, 2 * EMBED), jnp.float32),
            pltpu.VMEM((HALF, 2 * EMBED), jnp.float32),
            pltpu.SemaphoreType.DMA,
            pltpu.SemaphoreType.DMA,
            pltpu.SemaphoreType.DMA,
            pltpu.SemaphoreType.DMA,
        ],
        compiler_params=pltpu.CompilerParams(use_tc_tiling_on_sc=False),
    )
    return k(x3, token_table)


B_BLK = 128
H_BLK = H_PAD


def _xpose_body(j_ref, post_ref, out_ref):
    for hh in range(HALF):
        posvec = jnp.concatenate(
            [post_ref[:, 2 * hh], post_ref[:, 2 * hh + 1]])
        out_ref[pl.ds(hh * 2 * EMBED, 2 * EMBED), :] = (
            j_ref[:, hh, :].T + posvec[:, None])


def _tc_xpose(jflat, post):
    return pl.pallas_call(
        _xpose_body,
        out_shape=jax.ShapeDtypeStruct((HALF * 2 * EMBED, BATCH), jnp.float32),
        grid=(BATCH // B_BLK,),
        in_specs=[
            pl.BlockSpec((B_BLK, H_BLK, 2 * EMBED), lambda i: (i, 0, 0)),
            pl.BlockSpec((EMBED, MAXLEN), lambda i: (0, 0)),
        ],
        out_specs=pl.BlockSpec((H_BLK * 2 * EMBED, B_BLK), lambda i: (0, i)),
    )(jflat, post)


@jax.jit
def kernel(x, token_table, pos_table):
    x3 = x.reshape(BATCH, 2, HALF).astype(jnp.int32)
    j = _wrapped(x3, token_table)
    out_t = _tc_xpose(j, pos_table.T)
    return jnp.transpose(out_t.reshape(MAXLEN, EMBED, BATCH), (2, 0, 1))


# speedup vs baseline: 1.5980x; 1.5485x over previous
import jax
import jax.numpy as jnp
from jax import lax
from jax.experimental import pallas as pl
from jax.experimental.pallas import tpu as pltpu
from jax.experimental.pallas import tpu_sc as plsc

BATCH = 1024
MAXLEN = 200
EMBED = 64
HALF = 100
H_PAD = 104

NUM_CORES = 2
NUM_SUBCORES = 16
NUM_WORKERS = NUM_CORES * NUM_SUBCORES
SEQS_PER_WORKER = BATCH // NUM_WORKERS
LANES = 16
NBUF = 2
NROUNDS = SEQS_PER_WORKER // NBUF


def _embed_kernel(x_hbm, tok_hbm, pos_hbm, out_hbm, idx_v, pos_v,
                  gbufs, wbufs, gsems, osems):
    wid = lax.axis_index("s") * NUM_CORES + lax.axis_index("c")
    base = wid * SEQS_PER_WORKER

    pltpu.sync_copy(x_hbm.at[pl.ds(base, SEQS_PER_WORKER)], idx_v)
    pltpu.sync_copy(pos_hbm, pos_v)

    def start_gather(s, b):
        pltpu.make_async_copy(
            tok_hbm.at[idx_v.at[s, 0]], gbufs[b].at[pl.ds(0, HALF)], gsems[b]
        ).start()
        pltpu.make_async_copy(
            tok_hbm.at[idx_v.at[s, 1]], gbufs[b].at[pl.ds(HALF, HALF)], gsems[b]
        ).start()

    def wait_gather(s, b):
        pltpu.make_async_copy(
            tok_hbm.at[idx_v.at[s, 0]], gbufs[b].at[pl.ds(0, HALF)], gsems[b]
        ).wait()
        pltpu.make_async_copy(
            tok_hbm.at[idx_v.at[s, 1]], gbufs[b].at[pl.ds(HALF, HALF)], gsems[b]
        ).wait()

    def add_pos(b):
        @pl.loop(0, HALF)
        def _(h):
            r = 2 * h
            for j in range(EMBED // LANES):
                c = pl.ds(j * LANES, LANES)
                cl = pl.ds(j * LANES + EMBED, LANES)
                wbufs[b][h, c] = gbufs[b][r, c] + pos_v[r, c]
                wbufs[b][h, cl] = gbufs[b][r + 1, c] + pos_v[r + 1, c]

    def start_write(s, b):
        pltpu.make_async_copy(
            wbufs[b], out_hbm.at[base + s, pl.ds(0, HALF)], osems[b]
        ).start()

    def wait_write(s, b):
        pltpu.make_async_copy(
            wbufs[b], out_hbm.at[base + s, pl.ds(0, HALF)], osems[b]
        ).wait()

    for b in range(NBUF):
        start_gather(b, b)

    for b in range(NBUF):
        wait_gather(b, b)
        add_pos(b)
        start_gather(NBUF + b, b)
        start_write(b, b)

    @pl.loop(1, NROUNDS - 1)
    def _(g):
        for b in range(NBUF):
            s = g * NBUF + b
            wait_gather(s, b)
            wait_write(s - NBUF, b)
            add_pos(b)
            start_gather(s + NBUF, b)
            start_write(s, b)

    for b in range(NBUF):
        s = (NROUNDS - 1) * NBUF + b
        wait_gather(s, b)
        wait_write(s - NBUF, b)
        add_pos(b)
        start_write(s, b)
    for b in range(NBUF):
        s = (NROUNDS - 1) * NBUF + b
        wait_write(s, b)


def _wrapped(x3, token_table, pos_table):
    mesh = plsc.VectorSubcoreMesh(core_axis_name="c", subcore_axis_name="s")
    vmem_rows = lambda: pltpu.VMEM((MAXLEN, EMBED), jnp.float32)

    def body(x_hbm, tok_hbm, pos_hbm, out_hbm, idx_v, pos_v,
             g0, g1, w0, w1, gs0, gs1, os0, os1):
        _embed_kernel(x_hbm, tok_hbm, pos_hbm, out_hbm, idx_v, pos_v,
                      (g0, g1), (w0, w1), (gs0, gs1), (os0, os1))

    k = pl.kernel(
        body,
        out_type=jax.ShapeDtypeStruct((BATCH, H_PAD, 2 * EMBED), jnp.float32),
        mesh=mesh,
        scratch_types=[
            pltpu.VMEM((SEQS_PER_WORKER, 2, HALF), jnp.int32),
            vmem_rows(), vmem_rows(), vmem_rows(),
            pltpu.VMEM((HALF, 2 * EMBED), jnp.float32),
            pltpu.VMEM((HALF, 2 * EMBED), jnp.float32),
            pltpu.SemaphoreType.DMA,
            pltpu.SemaphoreType.DMA,
            pltpu.SemaphoreType.DMA,
            pltpu.SemaphoreType.DMA,
        ],
        compiler_params=pltpu.CompilerParams(use_tc_tiling_on_sc=False),
    )
    return k(x3, token_table, pos_table)


B_BLK = 128
H_BLK = H_PAD


def _xpose_body(j_ref, out_ref):
    for hh in range(HALF):
        out_ref[pl.ds(hh * 2 * EMBED, 2 * EMBED), :] = j_ref[:, hh, :].T


def _tc_xpose(jflat):
    return pl.pallas_call(
        _xpose_body,
        out_shape=jax.ShapeDtypeStruct((HALF * 2 * EMBED, BATCH), jnp.float32),
        grid=(BATCH // B_BLK,),
        in_specs=[
            pl.BlockSpec((B_BLK, H_BLK, 2 * EMBED), lambda i: (i, 0, 0)),
        ],
        out_specs=pl.BlockSpec((H_BLK * 2 * EMBED, B_BLK), lambda i: (0, i)),
    )(jflat)


@jax.jit
def kernel(x, token_table, pos_table):
    x3 = x.reshape(BATCH, 2, HALF).astype(jnp.int32)
    j = _wrapped(x3, token_table, pos_table)
    out_t = _tc_xpose(j)
    return jnp.transpose(out_t.reshape(MAXLEN, EMBED, BATCH), (2, 0, 1))
